# Initial kernel scaffold; baseline (speedup 1.0000x reference)
import jax
import jax.numpy as jnp
from jax.experimental import pallas as pl


def kernel(x, edge_attr, atom_index, e_idx, global_state, W_edge, b_edge, W_e, b_e):
    raise NotImplementedError("write your pallas kernel here")


# speedup vs baseline: 1.9214x; 1.9214x over previous
import functools

import jax
import jax.numpy as jnp
from jax.experimental import pallas as pl


def _mm_bias_body(x_ref, w_ref, b_ref, o_ref):
    o_ref[...] = (
        jnp.dot(x_ref[...], w_ref[...], preferred_element_type=jnp.float32)
        + b_ref[...]
    )


def _edge_proj_body(ea_ref, w1_ref, w2_ref, b_ref, pe_ref, base_ref):
    ea = ea_ref[...]
    pe_ref[...] = jnp.dot(ea, w1_ref[...], preferred_element_type=jnp.float32)
    z = jnp.dot(ea, w2_ref[...], preferred_element_type=jnp.float32) + b_ref[...]
    base_ref[...] = jnp.where(z > 0, z, jnp.exp(z) - 1.0)


def _node_proj(xg_pad, w_pad, b):
    n, k = xg_pad.shape
    blk = 2000
    return pl.pallas_call(
        _mm_bias_body,
        grid=(n // blk,),
        in_specs=[
            pl.BlockSpec((blk, k), lambda i: (i, 0)),
            pl.BlockSpec((k, 128), lambda i: (0, 0)),
            pl.BlockSpec((1, 128), lambda i: (0, 0)),
        ],
        out_specs=pl.BlockSpec((blk, 128), lambda i: (i, 0)),
        out_shape=jax.ShapeDtypeStruct((n, 128), jnp.float32),
    )(xg_pad, w_pad, b.reshape(1, 128))


def _edge_proj(edge_attr, w1, w2, b):
    e, k = edge_attr.shape
    blk = 2000
    return pl.pallas_call(
        _edge_proj_body,
        grid=(e // blk,),
        in_specs=[
            pl.BlockSpec((blk, k), lambda i: (i, 0)),
            pl.BlockSpec((k, 128), lambda i: (0, 0)),
            pl.BlockSpec((k, 128), lambda i: (0, 0)),
            pl.BlockSpec((1, 128), lambda i: (0, 0)),
        ],
        out_specs=[
            pl.BlockSpec((blk, 128), lambda i: (i, 0)),
            pl.BlockSpec((blk, 128), lambda i: (i, 0)),
        ],
        out_shape=[
            jax.ShapeDtypeStruct((e, 128), jnp.float32),
            jax.ShapeDtypeStruct((e, 128), jnp.float32),
        ],
    )(edge_attr, w1, w2, b.reshape(1, 128))


def kernel(x, edge_attr, atom_index, e_idx, global_state, W_edge, b_edge, W_e, b_e):
    e = edge_attr.shape[0]
    xg = jnp.concatenate([x, global_state], axis=1)
    k_node = xg.shape[1]
    k_pad = 64
    xg_pad = jnp.pad(xg, ((0, 0), (0, k_pad - k_node)))
    w_node = jnp.pad(W_edge[edge_attr.shape[1]:], ((0, k_pad - k_node), (0, 0)))

    p_node = _node_proj(xg_pad, w_node, b_edge)
    p_edge, base = _edge_proj(edge_attr, W_edge[:edge_attr.shape[1]], W_e, b_e)

    t = jax.nn.elu(jnp.take(p_edge, e_idx[1], axis=0)
                   + jnp.take(p_node, atom_index[0], axis=0))
    out = base + jax.ops.segment_sum(t, e_idx[0], num_segments=e)
    return out
